# baseline (device time: 69886 ns/iter reference)
import jax
import jax.numpy as jnp
from jax import lax
from jax.experimental import pallas as pl
from jax.experimental.pallas import tpu as pltpu

N_DEV = 8


def kernel(x, w_mat):
    m_total, k_per = x.shape
    k_total, n = w_mat.shape
    m_per = m_total // N_DEV

    x = x.astype(jnp.bfloat16)
    w = w_mat.astype(jnp.bfloat16)

    def body(x_ref, w_ref, out_ref, recv_buf, send_sems, recv_sems):
        my = lax.axis_index("i")

        barrier_sem = pltpu.get_barrier_semaphore()
        for d in range(1, N_DEV):
            peer = lax.rem(my + d, N_DEV)
            pl.semaphore_signal(
                barrier_sem, inc=1,
                device_id=(peer,), device_id_type=pl.DeviceIdType.MESH,
            )
        pl.semaphore_wait(barrier_sem, N_DEV - 1)

        rdmas = []
        for d in range(1, N_DEV):
            dst = lax.rem(my + d, N_DEV)
            rdma = pltpu.make_async_remote_copy(
                src_ref=x_ref.at[pl.ds(dst * m_per, m_per), :],
                dst_ref=recv_buf.at[d - 1],
                send_sem=send_sems.at[d - 1],
                recv_sem=recv_sems.at[d - 1],
                device_id=(dst,),
                device_id_type=pl.DeviceIdType.MESH,
            )
            rdma.start()
            rdmas.append(rdma)

        out_ref[:, :] = jnp.dot(
            x_ref[pl.ds(my * m_per, m_per), :],
            w_ref[pl.ds(my * k_per, k_per), :],
            preferred_element_type=jnp.float32,
        )

        for d in range(1, N_DEV):
            rdmas[d - 1].wait_recv()
            src = lax.rem(my - d + N_DEV, N_DEV)
            out_ref[:, :] += jnp.dot(
                recv_buf[d - 1],
                w_ref[pl.ds(src * k_per, k_per), :],
                preferred_element_type=jnp.float32,
            )

        for d in range(1, N_DEV):
            rdmas[d - 1].wait_send()

        y = out_ref[:, :]
        out_ref[:, :] = y / (1.0 + jnp.exp(-y))

    return pl.pallas_call(
        body,
        out_shape=jax.ShapeDtypeStruct((m_per, n), jnp.float32),
        in_specs=[
            pl.BlockSpec(memory_space=pltpu.VMEM),
            pl.BlockSpec(memory_space=pltpu.VMEM),
        ],
        out_specs=pl.BlockSpec(memory_space=pltpu.VMEM),
        scratch_shapes=[
            pltpu.VMEM((N_DEV - 1, m_per, k_per), jnp.bfloat16),
            pltpu.SemaphoreType.DMA((N_DEV - 1,)),
            pltpu.SemaphoreType.DMA((N_DEV - 1,)),
        ],
        compiler_params=pltpu.CompilerParams(collective_id=0),
    )(x, w)


# device time: 60075 ns/iter; 1.1633x vs baseline; 1.1633x over previous
import jax
import jax.numpy as jnp
from jax import lax
from jax.experimental import pallas as pl
from jax.experimental.pallas import tpu as pltpu

N_DEV = 8


def kernel(x, w_mat):
    m_total, k_per = x.shape
    k_total, n = w_mat.shape
    m_per = m_total // N_DEV

    def body(x_ref, w_ref, out_ref, x_bf, recv_buf, send_sems, recv_sems):
        my = lax.axis_index("i")

        barrier_sem = pltpu.get_barrier_semaphore()
        for d in range(1, N_DEV):
            peer = lax.rem(my + d, N_DEV)
            pl.semaphore_signal(
                barrier_sem, inc=1,
                device_id=(peer,), device_id_type=pl.DeviceIdType.MESH,
            )

        x_bf[:, :] = x_ref[:, :].astype(jnp.bfloat16)

        pl.semaphore_wait(barrier_sem, N_DEV - 1)

        rdmas = []
        for d in range(1, N_DEV):
            dst = lax.rem(my + d, N_DEV)
            rdma = pltpu.make_async_remote_copy(
                src_ref=x_bf.at[pl.ds(dst * m_per, m_per), :],
                dst_ref=recv_buf.at[d - 1],
                send_sem=send_sems.at[d - 1],
                recv_sem=recv_sems.at[d - 1],
                device_id=(dst,),
                device_id_type=pl.DeviceIdType.MESH,
            )
            rdma.start()
            rdmas.append(rdma)

        out_ref[:, :] = jnp.dot(
            x_bf[pl.ds(my * m_per, m_per), :],
            w_ref[pl.ds(my * k_per, k_per), :].astype(jnp.bfloat16),
            preferred_element_type=jnp.float32,
        )

        for d in range(1, N_DEV):
            rdmas[d - 1].wait_recv()
            src = lax.rem(my - d + N_DEV, N_DEV)
            out_ref[:, :] += jnp.dot(
                recv_buf[d - 1],
                w_ref[pl.ds(src * k_per, k_per), :].astype(jnp.bfloat16),
                preferred_element_type=jnp.float32,
            )

        for d in range(1, N_DEV):
            rdmas[d - 1].wait_send()

        y = out_ref[:, :]
        out_ref[:, :] = y / (1.0 + jnp.exp(-y))

    return pl.pallas_call(
        body,
        out_shape=jax.ShapeDtypeStruct((m_per, n), jnp.float32),
        in_specs=[
            pl.BlockSpec(memory_space=pltpu.VMEM),
            pl.BlockSpec(memory_space=pltpu.VMEM),
        ],
        out_specs=pl.BlockSpec(memory_space=pltpu.VMEM),
        scratch_shapes=[
            pltpu.VMEM((m_total, k_per), jnp.bfloat16),
            pltpu.VMEM((N_DEV - 1, m_per, k_per), jnp.bfloat16),
            pltpu.SemaphoreType.DMA((N_DEV - 1,)),
            pltpu.SemaphoreType.DMA((N_DEV - 1,)),
        ],
        compiler_params=pltpu.CompilerParams(
            collective_id=0, vmem_limit_bytes=100 * 1024 * 1024
        ),
    )(x, w_mat)


# device time: 48809 ns/iter; 1.4318x vs baseline; 1.2308x over previous
import jax
import jax.numpy as jnp
from jax import lax
from jax.experimental import pallas as pl
from jax.experimental.pallas import tpu as pltpu

N_DEV = 8


def kernel(x, w_mat):
    m_total, k_per = x.shape
    k_total, n = w_mat.shape
    m_per = m_total // N_DEV

    def body(
        x_hbm, w_hbm, out_ref,
        x_vm, x_bf, w_vm, recv_buf,
        send_sems, recv_sems, ready_sems, sem_x, sem_w,
    ):
        my = lax.axis_index("i")

        barrier_sem = pltpu.get_barrier_semaphore()
        pl.semaphore_signal(
            barrier_sem, inc=1,
            device_id=(my,), device_id_type=pl.DeviceIdType.MESH,
        )
        pl.semaphore_wait(barrier_sem, 1)

        cp_x = pltpu.make_async_copy(x_hbm, x_vm, sem_x)
        cp_x.start()
        cp_w = pltpu.make_async_copy(w_hbm, w_vm, sem_w)
        cp_w.start()

        for d in range(1, N_DEV):
            peer = lax.rem(my + d, N_DEV)
            pl.semaphore_signal(
                ready_sems.at[N_DEV - 1 - d], inc=1,
                device_id=(peer,), device_id_type=pl.DeviceIdType.MESH,
            )

        cp_x.wait()
        x_bf[:, :] = x_vm[:, :].astype(jnp.bfloat16)

        rdmas = []
        for d in range(1, N_DEV):
            dst = lax.rem(my + d, N_DEV)
            pl.semaphore_wait(ready_sems.at[d - 1], 1)
            rdma = pltpu.make_async_remote_copy(
                src_ref=x_bf.at[pl.ds(dst * m_per, m_per), :],
                dst_ref=recv_buf.at[d - 1],
                send_sem=send_sems.at[d - 1],
                recv_sem=recv_sems.at[d - 1],
                device_id=(dst,),
                device_id_type=pl.DeviceIdType.MESH,
            )
            rdma.start()
            rdmas.append(rdma)

        cp_w.wait()
        out_ref[:, :] = jnp.dot(
            x_bf[pl.ds(my * m_per, m_per), :],
            w_vm[pl.ds(my * k_per, k_per), :].astype(jnp.bfloat16),
            preferred_element_type=jnp.float32,
        )

        for d in range(1, N_DEV):
            rdmas[d - 1].wait_recv()
            src = lax.rem(my - d + N_DEV, N_DEV)
            out_ref[:, :] += jnp.dot(
                recv_buf[d - 1],
                w_vm[pl.ds(src * k_per, k_per), :].astype(jnp.bfloat16),
                preferred_element_type=jnp.float32,
            )

        for d in range(1, N_DEV):
            rdmas[d - 1].wait_send()

        y = out_ref[:, :]
        out_ref[:, :] = y / (1.0 + jnp.exp(-y))

    return pl.pallas_call(
        body,
        out_shape=jax.ShapeDtypeStruct((m_per, n), jnp.float32),
        in_specs=[
            pl.BlockSpec(memory_space=pltpu.MemorySpace.HBM),
            pl.BlockSpec(memory_space=pltpu.MemorySpace.HBM),
        ],
        out_specs=pl.BlockSpec(memory_space=pltpu.VMEM),
        scratch_shapes=[
            pltpu.VMEM((m_total, k_per), jnp.float32),
            pltpu.VMEM((m_total, k_per), jnp.bfloat16),
            pltpu.VMEM((k_total, n), jnp.float32),
            pltpu.VMEM((N_DEV - 1, m_per, k_per), jnp.bfloat16),
            pltpu.SemaphoreType.DMA((N_DEV - 1,)),
            pltpu.SemaphoreType.DMA((N_DEV - 1,)),
            pltpu.SemaphoreType.REGULAR((N_DEV - 1,)),
            pltpu.SemaphoreType.DMA,
            pltpu.SemaphoreType.DMA,
        ],
        compiler_params=pltpu.CompilerParams(
            collective_id=0, vmem_limit_bytes=100 * 1024 * 1024
        ),
    )(x, w_mat)
